# Initial kernel scaffold; baseline (speedup 1.0000x reference)
import jax
import jax.numpy as jnp
from jax.experimental import pallas as pl


def kernel(x, edge_index, edge_attr, W1e, b1e, W2e, b2e, W1l, b1l, W2l, b2l):
    raise NotImplementedError("write your pallas kernel here")


# speedup vs baseline: 2.9136x; 2.9136x over previous
import functools

import jax
import jax.numpy as jnp
from jax import lax
from jax.experimental import pallas as pl
from jax.experimental.pallas import tpu as pltpu
from jax.experimental.pallas import tpu_sc as plsc

N_SEND = 12288
N_REC = 49152
K = 4
E = N_REC * K
D = 128
EDGE_IN = 4
LIN_IN = 2 * D

NC = 2
NS = 16
NW = NC * NS
R_PER_W = N_REC // NW
CHUNK = 128
N_STEPS = R_PER_W // CHUNK


def _sc_gather_body(x_hbm, senders_t_hbm, out_hbm, idx_v, buf0, buf1, buf2,
                    buf3, sem):
    bufs = (buf0, buf1, buf2, buf3)
    wid = lax.axis_index("s") * NC + lax.axis_index("c")
    base = wid * R_PER_W

    def step(c, carry):
        rbase = base + c * CHUNK
        pltpu.sync_copy(senders_t_hbm.at[:, pl.ds(rbase, CHUNK)], idx_v)
        descs = [pltpu.async_copy(x_hbm.at[idx_v.at[k]], bufs[k], sem)
                 for k in range(K)]
        for d in descs:
            d.wait()
        for k in range(K):
            pltpu.sync_copy(bufs[k], out_hbm.at[k, pl.ds(rbase, CHUNK)])
        return carry

    lax.fori_loop(0, N_STEPS, step, 0)


@functools.cache
def _get_sc_gather():
    return pl.kernel(
        _sc_gather_body,
        out_type=jax.ShapeDtypeStruct((K, N_REC, D), jnp.float32),
        mesh=plsc.VectorSubcoreMesh(core_axis_name="c", subcore_axis_name="s",
                                    num_cores=NC, num_subcores=NS),
        scratch_types=[
            pltpu.VMEM((K, CHUNK), jnp.int32),
            pltpu.VMEM((CHUNK, D), jnp.float32),
            pltpu.VMEM((CHUNK, D), jnp.float32),
            pltpu.VMEM((CHUNK, D), jnp.float32),
            pltpu.VMEM((CHUNK, D), jnp.float32),
            pltpu.SemaphoreType.DMA,
        ],
    )


R_TILE = 1024


def _tc_body(gt_ref, ea_ref, w1e_ref, b1e_ref, w2e_ref, b2e_ref, w1l_ref,
             b1l_ref, w2l_ref, b2l_ref, out_ref):
    f32 = jnp.float32
    gsum = gt_ref[0] + gt_ref[1] + gt_ref[2] + gt_ref[3]
    hsum = jnp.maximum(
        jnp.dot(ea_ref[0], w1e_ref[...], preferred_element_type=f32)
        + b1e_ref[...], 0.0)
    for k in range(1, K):
        hsum += jnp.maximum(
            jnp.dot(ea_ref[k], w1e_ref[...], preferred_element_type=f32)
            + b1e_ref[...], 0.0)
    ef = jnp.dot(hsum, w2e_ref[...], preferred_element_type=f32) \
        + float(K) * b2e_ref[...]
    g = jnp.maximum(
        jnp.dot(gsum, w1l_ref[0:D, :], preferred_element_type=f32)
        + jnp.dot(ef, w1l_ref[D:LIN_IN, :], preferred_element_type=f32)
        + b1l_ref[...], 0.0)
    out_ref[...] = jnp.dot(g, w2l_ref[...], preferred_element_type=f32) \
        + b2l_ref[...]


def _tc_call(gt, ea_t, W1e, b1e, W2e, b2e, W1l, b1l, W2l, b2l):
    grid = (N_REC // R_TILE,)
    full = lambda shape: pl.BlockSpec(shape, lambda i: (0,) * len(shape))
    return pl.pallas_call(
        _tc_body,
        grid=grid,
        in_specs=[
            pl.BlockSpec((K, R_TILE, D), lambda i: (0, i, 0)),
            pl.BlockSpec((K, R_TILE, EDGE_IN), lambda i: (0, i, 0)),
            full((EDGE_IN, D)),
            full((1, D)),
            full((D, D)),
            full((1, D)),
            full((LIN_IN, D)),
            full((1, D)),
            full((D, D)),
            full((1, D)),
        ],
        out_specs=pl.BlockSpec((R_TILE, D), lambda i: (i, 0)),
        out_shape=jax.ShapeDtypeStruct((N_REC, D), jnp.float32),
    )(gt, ea_t, W1e, b1e, W2e, b2e, W1l, b1l, W2l, b2l)


def kernel(x, edge_index, edge_attr, W1e, b1e, W2e, b2e, W1l, b1l, W2l, b2l):
    x2d = x.reshape(N_SEND, D)
    senders_t = edge_index[0].reshape(N_REC, K).T
    ea_t = edge_attr.reshape(N_REC, K, EDGE_IN).transpose(1, 0, 2)
    gt = _get_sc_gather()(x2d, senders_t)
    out = _tc_call(gt, ea_t, W1e, b1e.reshape(1, D), W2e, b2e.reshape(1, D),
                   W1l, b1l.reshape(1, D), W2l, b2l.reshape(1, D))
    return out.reshape(1, N_REC, D)


# speedup vs baseline: 6.5778x; 2.2576x over previous
import functools

import jax
import jax.numpy as jnp
from jax import lax
from jax.experimental import pallas as pl
from jax.experimental.pallas import tpu as pltpu
from jax.experimental.pallas import tpu_sc as plsc

N_SEND = 12288
N_REC = 49152
K = 4
E = N_REC * K
D = 128
EDGE_IN = 4
LIN_IN = 2 * D

NC = 2
NS = 16
NW = NC * NS
R_PER_W = N_REC // NW
CHUNK = 128
N_STEPS = R_PER_W // CHUNK


def _sc_gather_body(x_hbm, senders_t_hbm, out_hbm, idx_v, buf0, buf1, buf2,
                    buf3, sem):
    bufs = (buf0, buf1, buf2, buf3)
    wid = lax.axis_index("s") * NC + lax.axis_index("c")
    base = wid * R_PER_W

    def step(c, carry):
        rbase = base + c * CHUNK
        pltpu.sync_copy(senders_t_hbm.at[:, pl.ds(rbase, CHUNK)], idx_v)
        descs = [pltpu.async_copy(x_hbm.at[idx_v.at[k]], bufs[k], sem)
                 for k in range(K)]
        for d in descs:
            d.wait()
        for k in range(K):
            pltpu.sync_copy(bufs[k], out_hbm.at[k, pl.ds(rbase, CHUNK)])
        return carry

    lax.fori_loop(0, N_STEPS, step, 0)


@functools.cache
def _get_sc_gather():
    return pl.kernel(
        _sc_gather_body,
        out_type=jax.ShapeDtypeStruct((K, N_REC, D), jnp.float32),
        mesh=plsc.VectorSubcoreMesh(core_axis_name="c", subcore_axis_name="s",
                                    num_cores=NC, num_subcores=NS),
        scratch_types=[
            pltpu.VMEM((K, CHUNK), jnp.int32),
            pltpu.VMEM((CHUNK, D), jnp.float32),
            pltpu.VMEM((CHUNK, D), jnp.float32),
            pltpu.VMEM((CHUNK, D), jnp.float32),
            pltpu.VMEM((CHUNK, D), jnp.float32),
            pltpu.SemaphoreType.DMA,
        ],
    )


R_TILE = 2048


def _tc_body(gt_ref, ea_ref, w1e_rep_ref, b1e_ref, w2e_ref, b2e_ref, w1l_ref,
             b1l_ref, w2l_ref, b2l_ref, out_ref):
    f32 = jnp.float32
    gsum = gt_ref[0] + gt_ref[1] + gt_ref[2] + gt_ref[3]
    hbig = jnp.dot(ea_ref[...], w1e_rep_ref[...],
                   preferred_element_type=f32)
    hsum = jnp.maximum(hbig[:, 0:D] + b1e_ref[...], 0.0)
    for k in range(1, K):
        hsum += jnp.maximum(hbig[:, k * D:(k + 1) * D] + b1e_ref[...], 0.0)
    ef = jnp.dot(hsum, w2e_ref[...], preferred_element_type=f32) \
        + float(K) * b2e_ref[...]
    g = jnp.maximum(
        jnp.dot(gsum, w1l_ref[0:D, :], preferred_element_type=f32)
        + jnp.dot(ef, w1l_ref[D:LIN_IN, :], preferred_element_type=f32)
        + b1l_ref[...], 0.0)
    out_ref[...] = jnp.dot(g, w2l_ref[...], preferred_element_type=f32) \
        + b2l_ref[...]


def _tc_call(gt, ea2, W1e_rep, b1e, W2e, b2e, W1l, b1l, W2l, b2l):
    grid = (N_REC // R_TILE,)
    full = lambda shape: pl.BlockSpec(shape, lambda i: (0,) * len(shape))
    return pl.pallas_call(
        _tc_body,
        grid=grid,
        in_specs=[
            pl.BlockSpec((K, R_TILE, D), lambda i: (0, i, 0)),
            pl.BlockSpec((R_TILE, K * EDGE_IN), lambda i: (i, 0)),
            full((K * EDGE_IN, K * D)),
            full((1, D)),
            full((D, D)),
            full((1, D)),
            full((LIN_IN, D)),
            full((1, D)),
            full((D, D)),
            full((1, D)),
        ],
        out_specs=pl.BlockSpec((R_TILE, D), lambda i: (i, 0)),
        out_shape=jax.ShapeDtypeStruct((N_REC, D), jnp.float32),
    )(gt, ea2, W1e_rep, b1e, W2e, b2e, W1l, b1l, W2l, b2l)


def kernel(x, edge_index, edge_attr, W1e, b1e, W2e, b2e, W1l, b1l, W2l, b2l):
    x2d = x.reshape(N_SEND, D)
    senders_t = edge_index[0].reshape(N_REC, K).T
    ea2 = edge_attr.reshape(N_REC, K * EDGE_IN)
    eye_blocks = jnp.eye(K, dtype=jnp.float32)
    W1e_rep = (eye_blocks[:, None, :, None]
               * W1e[None, :, None, :]).reshape(K * EDGE_IN, K * D)
    gt = _get_sc_gather()(x2d, senders_t)
    out = _tc_call(gt, ea2, W1e_rep, b1e.reshape(1, D), W2e,
                   b2e.reshape(1, D), W1l, b1l.reshape(1, D), W2l,
                   b2l.reshape(1, D))
    return out.reshape(1, N_REC, D)


# speedup vs baseline: 7.5097x; 1.1417x over previous
import functools

import jax
import jax.numpy as jnp
from jax import lax
from jax.experimental import pallas as pl
from jax.experimental.pallas import tpu as pltpu
from jax.experimental.pallas import tpu_sc as plsc

N_SEND = 12288
N_REC = 49152
K = 4
E = N_REC * K
D = 128
EDGE_IN = 4
LIN_IN = 2 * D

NC = 2
NS = 16
NW = NC * NS
R_PER_W = N_REC // NW
CHUNK = 64


N_CHUNKS = R_PER_W // CHUNK
LANES = 16


def _sc_gather_body(x_hbm, senders_t_hbm, out_hbm, idx_all, bufs0, bufs1,
                    osum0, osum1, sem_g0, sem_g1, sem_o0, sem_o1):
    bufs = (bufs0, bufs1)
    osum = (osum0, osum1)
    sem_g = (sem_g0, sem_g1)
    sem_o = (sem_o0, sem_o1)
    wid = lax.axis_index("s") * NC + lax.axis_index("c")
    base = wid * R_PER_W

    pltpu.sync_copy(senders_t_hbm.at[:, pl.ds(base, R_PER_W)], idx_all)

    def issue_gathers(c, s):
        for k in range(K):
            pltpu.async_copy(
                x_hbm.at[idx_all.at[k, pl.ds(c * CHUNK, CHUNK)]],
                bufs[s].at[k], sem_g[s])

    def wait_gathers(s):
        for k in range(K):
            pltpu.make_async_copy(x_hbm.at[pl.ds(0, CHUNK)], bufs[s].at[k],
                                  sem_g[s]).wait()

    def wait_out(s):
        pltpu.make_async_copy(osum[s], out_hbm.at[pl.ds(0, CHUNK)],
                              sem_o[s]).wait()

    def reduce_k(s):
        b = bufs[s]
        o = osum[s]

        def row(r, carry):
            for col in range(D // LANES):
                sl = pl.ds(col * LANES, LANES)
                o[r, sl] = ((b[0, r, sl] + b[1, r, sl])
                            + (b[2, r, sl] + b[3, r, sl]))
            return carry

        lax.fori_loop(0, CHUNK, row, 0)

    def out_copy(c, s):
        pltpu.async_copy(osum[s], out_hbm.at[pl.ds(base + c * CHUNK, CHUNK)],
                         sem_o[s])

    issue_gathers(0, 0)

    def body(i, carry):
        ca = 2 * i
        cb = 2 * i + 1
        issue_gathers(cb, 1)
        wait_gathers(0)

        @pl.when(i > 0)
        def _():
            wait_out(0)

        reduce_k(0)
        out_copy(ca, 0)

        @pl.when(i < N_CHUNKS // 2 - 1)
        def _():
            issue_gathers(ca + 2, 0)

        wait_gathers(1)

        @pl.when(i > 0)
        def _():
            wait_out(1)

        reduce_k(1)
        out_copy(cb, 1)
        return carry

    lax.fori_loop(0, N_CHUNKS // 2, body, 0)
    wait_out(0)
    wait_out(1)


@functools.cache
def _get_sc_gather():
    return pl.kernel(
        _sc_gather_body,
        out_type=jax.ShapeDtypeStruct((N_REC, D), jnp.float32),
        mesh=plsc.VectorSubcoreMesh(core_axis_name="c", subcore_axis_name="s",
                                    num_cores=NC, num_subcores=NS),
        scratch_types=[
            pltpu.VMEM((K, R_PER_W), jnp.int32),
            pltpu.VMEM((K, CHUNK, D), jnp.float32),
            pltpu.VMEM((K, CHUNK, D), jnp.float32),
            pltpu.VMEM((CHUNK, D), jnp.float32),
            pltpu.VMEM((CHUNK, D), jnp.float32),
            pltpu.SemaphoreType.DMA,
            pltpu.SemaphoreType.DMA,
            pltpu.SemaphoreType.DMA,
            pltpu.SemaphoreType.DMA,
        ],
    )


R_TILE = 2048


def _tc_body(xs_ref, ea_ref, w1e_rep_ref, b1e_ref, w2e_ref, b2e_ref, w1l_ref,
             b1l_ref, w2l_ref, b2l_ref, out_ref):
    f32 = jnp.float32
    gsum = xs_ref[...]
    hbig = jnp.dot(ea_ref[...], w1e_rep_ref[...],
                   preferred_element_type=f32)
    hsum = jnp.maximum(hbig[:, 0:D] + b1e_ref[...], 0.0)
    for k in range(1, K):
        hsum += jnp.maximum(hbig[:, k * D:(k + 1) * D] + b1e_ref[...], 0.0)
    ef = jnp.dot(hsum, w2e_ref[...], preferred_element_type=f32) \
        + float(K) * b2e_ref[...]
    g = jnp.maximum(
        jnp.dot(gsum, w1l_ref[0:D, :], preferred_element_type=f32)
        + jnp.dot(ef, w1l_ref[D:LIN_IN, :], preferred_element_type=f32)
        + b1l_ref[...], 0.0)
    out_ref[...] = jnp.dot(g, w2l_ref[...], preferred_element_type=f32) \
        + b2l_ref[...]


def _tc_call(xs, ea2, W1e_rep, b1e, W2e, b2e, W1l, b1l, W2l, b2l):
    grid = (N_REC // R_TILE,)
    full = lambda shape: pl.BlockSpec(shape, lambda i: (0,) * len(shape))
    return pl.pallas_call(
        _tc_body,
        grid=grid,
        in_specs=[
            pl.BlockSpec((R_TILE, D), lambda i: (i, 0)),
            pl.BlockSpec((R_TILE, K * EDGE_IN), lambda i: (i, 0)),
            full((K * EDGE_IN, K * D)),
            full((1, D)),
            full((D, D)),
            full((1, D)),
            full((LIN_IN, D)),
            full((1, D)),
            full((D, D)),
            full((1, D)),
        ],
        out_specs=pl.BlockSpec((R_TILE, D), lambda i: (i, 0)),
        out_shape=jax.ShapeDtypeStruct((N_REC, D), jnp.float32),
    )(xs, ea2, W1e_rep, b1e, W2e, b2e, W1l, b1l, W2l, b2l)


def kernel(x, edge_index, edge_attr, W1e, b1e, W2e, b2e, W1l, b1l, W2l, b2l):
    x2d = x.reshape(N_SEND, D)
    senders_t = edge_index[0].reshape(N_REC, K).T
    ea2 = edge_attr.reshape(N_REC, K * EDGE_IN)
    eye_blocks = jnp.eye(K, dtype=jnp.float32)
    W1e_rep = (eye_blocks[:, None, :, None]
               * W1e[None, :, None, :]).reshape(K * EDGE_IN, K * D)
    xs = _get_sc_gather()(x2d, senders_t)
    out = _tc_call(xs, ea2, W1e_rep, b1e.reshape(1, D), W2e,
                   b2e.reshape(1, D), W1l, b1l.reshape(1, D), W2l,
                   b2l.reshape(1, D))
    return out.reshape(1, N_REC, D)


# speedup vs baseline: 9.8603x; 1.3130x over previous
import functools

import jax
import jax.numpy as jnp
from jax import lax
from jax.experimental import pallas as pl
from jax.experimental.pallas import tpu as pltpu
from jax.experimental.pallas import tpu_sc as plsc

N_SEND = 12288
N_REC = 49152
K = 4
E = N_REC * K
D = 128
EDGE_IN = 4
LIN_IN = 2 * D

NC = 2
NS = 16
NW = NC * NS
R_PER_W = N_REC // NW
CHUNK = 64
EC = CHUNK * K

N_CHUNKS = R_PER_W // CHUNK
LANES = 16


def _sc_gather_body(x_hbm, edge_index_hbm, out_hbm, idx_all, bufs0, bufs1,
                    osum0, osum1, sem_g0, sem_g1, sem_o0, sem_o1):
    bufs = (bufs0, bufs1)
    osum = (osum0, osum1)
    sem_g = (sem_g0, sem_g1)
    sem_o = (sem_o0, sem_o1)
    wid = lax.axis_index("s") * NC + lax.axis_index("c")
    base = wid * R_PER_W

    pltpu.sync_copy(edge_index_hbm.at[0, pl.ds(base * K, R_PER_W * K)],
                    idx_all)

    def issue_gather(c, s):
        pltpu.async_copy(x_hbm.at[idx_all.at[pl.ds(c * EC, EC)]], bufs[s],
                         sem_g[s])

    def wait_gather(s):
        pltpu.make_async_copy(x_hbm.at[pl.ds(0, EC)], bufs[s],
                              sem_g[s]).wait()

    def wait_out(s):
        pltpu.make_async_copy(osum[s], out_hbm.at[pl.ds(0, CHUNK)],
                              sem_o[s]).wait()

    def reduce_k(s):
        b = bufs[s]
        o = osum[s]

        def row(r, carry):
            e = r * K
            for col in range(D // LANES):
                sl = pl.ds(col * LANES, LANES)
                o[r, sl] = ((b[e, sl] + b[e + 1, sl])
                            + (b[e + 2, sl] + b[e + 3, sl]))
            return carry

        lax.fori_loop(0, CHUNK, row, 0)

    def out_copy(c, s):
        pltpu.async_copy(osum[s], out_hbm.at[pl.ds(base + c * CHUNK, CHUNK)],
                         sem_o[s])

    issue_gather(0, 0)

    def body(i, carry):
        ca = 2 * i
        cb = 2 * i + 1
        issue_gather(cb, 1)
        wait_gather(0)

        @pl.when(i > 0)
        def _():
            wait_out(0)

        reduce_k(0)
        out_copy(ca, 0)

        @pl.when(i < N_CHUNKS // 2 - 1)
        def _():
            issue_gather(ca + 2, 0)

        wait_gather(1)

        @pl.when(i > 0)
        def _():
            wait_out(1)

        reduce_k(1)
        out_copy(cb, 1)
        return carry

    lax.fori_loop(0, N_CHUNKS // 2, body, 0)
    wait_out(0)
    wait_out(1)


@functools.cache
def _get_sc_gather():
    return pl.kernel(
        _sc_gather_body,
        out_type=jax.ShapeDtypeStruct((N_REC, D), jnp.float32),
        mesh=plsc.VectorSubcoreMesh(core_axis_name="c", subcore_axis_name="s",
                                    num_cores=NC, num_subcores=NS),
        scratch_types=[
            pltpu.VMEM((R_PER_W * K,), jnp.int32),
            pltpu.VMEM((EC, D), jnp.float32),
            pltpu.VMEM((EC, D), jnp.float32),
            pltpu.VMEM((CHUNK, D), jnp.float32),
            pltpu.VMEM((CHUNK, D), jnp.float32),
            pltpu.SemaphoreType.DMA,
            pltpu.SemaphoreType.DMA,
            pltpu.SemaphoreType.DMA,
            pltpu.SemaphoreType.DMA,
        ],
    )


R_TILE = 2048


def _tc_body(xs_ref, eat_ref, w1e_ref, b1e_ref, w2e_ref, b2e_ref, w1l_ref,
             b1l_ref, w2l_ref, b2l_ref, out_ref):
    f32 = jnp.float32
    gsum = xs_ref[...]
    hraw = lax.dot_general(eat_ref[...], w1e_ref[...],
                           (((0,), (0,)), ((), ())),
                           preferred_element_type=f32)
    h = jnp.maximum(hraw + b1e_ref[...], 0.0)
    h3 = h.reshape(R_TILE, K, D)
    hsum = (h3[:, 0, :] + h3[:, 1, :]) + (h3[:, 2, :] + h3[:, 3, :])
    ef = jnp.dot(hsum, w2e_ref[...], preferred_element_type=f32) \
        + float(K) * b2e_ref[...]
    g = jnp.maximum(
        jnp.dot(gsum, w1l_ref[0:D, :], preferred_element_type=f32)
        + jnp.dot(ef, w1l_ref[D:LIN_IN, :], preferred_element_type=f32)
        + b1l_ref[...], 0.0)
    out_ref[...] = jnp.dot(g, w2l_ref[...], preferred_element_type=f32) \
        + b2l_ref[...]


def _tc_call(xs, ea_t, W1e, b1e, W2e, b2e, W1l, b1l, W2l, b2l):
    grid = (N_REC // R_TILE,)
    full = lambda shape: pl.BlockSpec(shape, lambda i: (0,) * len(shape))
    return pl.pallas_call(
        _tc_body,
        grid=grid,
        in_specs=[
            pl.BlockSpec((R_TILE, D), lambda i: (i, 0)),
            pl.BlockSpec((EDGE_IN, R_TILE * K), lambda i: (0, i)),
            full((EDGE_IN, D)),
            full((1, D)),
            full((D, D)),
            full((1, D)),
            full((LIN_IN, D)),
            full((1, D)),
            full((D, D)),
            full((1, D)),
        ],
        out_specs=pl.BlockSpec((R_TILE, D), lambda i: (i, 0)),
        out_shape=jax.ShapeDtypeStruct((N_REC, D), jnp.float32),
    )(xs, ea_t, W1e, b1e, W2e, b2e, W1l, b1l, W2l, b2l)


def kernel(x, edge_index, edge_attr, W1e, b1e, W2e, b2e, W1l, b1l, W2l, b2l):
    x2d = x.reshape(N_SEND, D)
    ea_t = edge_attr.T
    xs = _get_sc_gather()(x2d, edge_index)
    out = _tc_call(xs, ea_t, W1e, b1e.reshape(1, D), W2e,
                   b2e.reshape(1, D), W1l, b1l.reshape(1, D), W2l,
                   b2l.reshape(1, D))
    return out.reshape(1, N_REC, D)


# speedup vs baseline: 12.6086x; 1.2787x over previous
import functools

import jax
import jax.numpy as jnp
from jax import lax
from jax.experimental import pallas as pl
from jax.experimental.pallas import tpu as pltpu
from jax.experimental.pallas import tpu_sc as plsc

N_SEND = 12288
N_REC = 49152
K = 4
E = N_REC * K
D = 128
EDGE_IN = 4
LIN_IN = 2 * D

NC = 2
NS = 16
NW = NC * NS
R_PER_W = N_REC // NW
CHUNK = 64
EC = CHUNK * K

N_CHUNKS = R_PER_W // CHUNK
LANES = 16


def _sc_gather_body(x_hbm, edge_index_hbm, out_hbm, idx_all, bufs0, bufs1,
                    osum0, osum1, sem_g0, sem_g1, sem_o0, sem_o1):
    bufs = (bufs0, bufs1)
    osum = (osum0, osum1)
    sem_g = (sem_g0, sem_g1)
    sem_o = (sem_o0, sem_o1)
    wid = lax.axis_index("s") * NC + lax.axis_index("c")
    base = wid * R_PER_W

    pltpu.sync_copy(edge_index_hbm.at[0, pl.ds(base * K, R_PER_W * K)],
                    idx_all)

    def issue_gather(c, s):
        pltpu.async_copy(x_hbm.at[idx_all.at[pl.ds(c * EC, EC)]], bufs[s],
                         sem_g[s])

    def wait_gather(s):
        pltpu.make_async_copy(x_hbm.at[pl.ds(0, EC)], bufs[s],
                              sem_g[s]).wait()

    def wait_out(s):
        pltpu.make_async_copy(osum[s], out_hbm.at[pl.ds(0, CHUNK)],
                              sem_o[s]).wait()

    def reduce_k(s):
        b = bufs[s]
        o = osum[s]

        def row(r, carry):
            e = r * K
            for col in range(D // LANES):
                sl = pl.ds(col * LANES, LANES)
                o[r, sl] = ((b[e, sl] + b[e + 1, sl])
                            + (b[e + 2, sl] + b[e + 3, sl]))
            return carry

        lax.fori_loop(0, CHUNK, row, 0)

    def out_copy(c, s):
        pltpu.async_copy(osum[s], out_hbm.at[pl.ds(base + c * CHUNK, CHUNK)],
                         sem_o[s])

    issue_gather(0, 0)

    def body(i, carry):
        ca = 2 * i
        cb = 2 * i + 1
        issue_gather(cb, 1)
        wait_gather(0)

        @pl.when(i > 0)
        def _():
            wait_out(0)

        reduce_k(0)
        out_copy(ca, 0)

        @pl.when(i < N_CHUNKS // 2 - 1)
        def _():
            issue_gather(ca + 2, 0)

        wait_gather(1)

        @pl.when(i > 0)
        def _():
            wait_out(1)

        reduce_k(1)
        out_copy(cb, 1)
        return carry

    lax.fori_loop(0, N_CHUNKS // 2, body, 0)
    wait_out(0)
    wait_out(1)


@functools.cache
def _get_sc_gather():
    return pl.kernel(
        _sc_gather_body,
        out_type=jax.ShapeDtypeStruct((N_REC, D), jnp.float32),
        mesh=plsc.VectorSubcoreMesh(core_axis_name="c", subcore_axis_name="s",
                                    num_cores=NC, num_subcores=NS),
        scratch_types=[
            pltpu.VMEM((R_PER_W * K,), jnp.int32),
            pltpu.VMEM((EC, D), jnp.float32),
            pltpu.VMEM((EC, D), jnp.float32),
            pltpu.VMEM((CHUNK, D), jnp.float32),
            pltpu.VMEM((CHUNK, D), jnp.float32),
            pltpu.SemaphoreType.DMA,
            pltpu.SemaphoreType.DMA,
            pltpu.SemaphoreType.DMA,
            pltpu.SemaphoreType.DMA,
        ],
    )


R_TILE = 2048


def _tc_edge_body(eat_ref, w1e_ref, b1e_ref, w2e_ref, b2e_ref, ef_ref):
    f32 = jnp.float32
    hraw = lax.dot_general(eat_ref[...], w1e_ref[...],
                           (((0,), (0,)), ((), ())),
                           preferred_element_type=f32)
    h = jnp.maximum(hraw + b1e_ref[...], 0.0)
    h3 = h.reshape(R_TILE, K, D)
    hsum = (h3[:, 0, :] + h3[:, 1, :]) + (h3[:, 2, :] + h3[:, 3, :])
    ef_ref[...] = jnp.dot(hsum, w2e_ref[...], preferred_element_type=f32) \
        + float(K) * b2e_ref[...]


def _tc_edge_call(ea_t, W1e, b1e, W2e, b2e):
    grid = (N_REC // R_TILE,)
    full = lambda shape: pl.BlockSpec(shape, lambda i: (0,) * len(shape))
    return pl.pallas_call(
        _tc_edge_body,
        grid=grid,
        in_specs=[
            pl.BlockSpec((EDGE_IN, R_TILE * K), lambda i: (0, i)),
            full((EDGE_IN, D)),
            full((1, D)),
            full((D, D)),
            full((1, D)),
        ],
        out_specs=pl.BlockSpec((R_TILE, D), lambda i: (i, 0)),
        out_shape=jax.ShapeDtypeStruct((N_REC, D), jnp.float32),
    )(ea_t, W1e, b1e, W2e, b2e)


def _tc_final_body(xs_ref, ef_ref, w1l_ref, b1l_ref, w2l_ref, b2l_ref,
                   out_ref):
    f32 = jnp.float32
    g = jnp.maximum(
        jnp.dot(xs_ref[...], w1l_ref[0:D, :], preferred_element_type=f32)
        + jnp.dot(ef_ref[...], w1l_ref[D:LIN_IN, :],
                  preferred_element_type=f32)
        + b1l_ref[...], 0.0)
    out_ref[...] = jnp.dot(g, w2l_ref[...], preferred_element_type=f32) \
        + b2l_ref[...]


def _tc_final_call(xs, ef, W1l, b1l, W2l, b2l):
    grid = (N_REC // R_TILE,)
    full = lambda shape: pl.BlockSpec(shape, lambda i: (0,) * len(shape))
    return pl.pallas_call(
        _tc_final_body,
        grid=grid,
        in_specs=[
            pl.BlockSpec((R_TILE, D), lambda i: (i, 0)),
            pl.BlockSpec((R_TILE, D), lambda i: (i, 0)),
            full((LIN_IN, D)),
            full((1, D)),
            full((D, D)),
            full((1, D)),
        ],
        out_specs=pl.BlockSpec((R_TILE, D), lambda i: (i, 0)),
        out_shape=jax.ShapeDtypeStruct((N_REC, D), jnp.float32),
    )(xs, ef, W1l, b1l, W2l, b2l)


def kernel(x, edge_index, edge_attr, W1e, b1e, W2e, b2e, W1l, b1l, W2l, b2l):
    x2d = x.reshape(N_SEND, D)
    ea_t = edge_attr.T
    xs = _get_sc_gather()(x2d, edge_index)
    ef = _tc_edge_call(ea_t, W1e, b1e.reshape(1, D), W2e, b2e.reshape(1, D))
    out = _tc_final_call(xs, ef, W1l, b1l.reshape(1, D), W2l,
                         b2l.reshape(1, D))
    return out.reshape(1, N_REC, D)
